# baseline (speedup 1.0000x reference)
import functools

import jax
import jax.numpy as jnp
from jax import lax
from jax.experimental import pallas as pl
from jax.experimental.pallas import tpu as pltpu
from jax.experimental.pallas import tpu_sc as plsc

N = 10000
E = 320000
D_IN = 128
D_HID = 128
D_LAT = 64

NC = 2
NS = 16
NW = NC * NS

N_PAD = 10240
ROWS_PER_TILE = N_PAD // NS
EPT = 10240
E_PAD = EPT * NW

_MESH = plsc.VectorSubcoreMesh(core_axis_name="c", subcore_axis_name="s")
_SC_PARAMS = pltpu.CompilerParams(
    needs_layout_passes=False, use_tc_tiling_on_sc=False
)


@functools.partial(
    pl.kernel,
    mesh=_MESH,
    out_type=jax.ShapeDtypeStruct((NC, N_PAD), jnp.float32),
    scratch_types=[
        pltpu.VMEM((EPT,), jnp.int32),
        pltpu.VMEM((N_PAD,), jnp.float32),
        pltpu.VMEM((ROWS_PER_TILE,), jnp.float32),
        pltpu.VMEM((ROWS_PER_TILE,), jnp.float32),
        pltpu.VMEM_SHARED((NS, N_PAD), jnp.float32),
    ],
    compiler_params=_SC_PARAMS,
)
def _deg_kernel(dst_hbm, out_hbm, dst_v, hist_v, tmp_v, acc_v, spart):
    c = lax.axis_index("c")
    s = lax.axis_index("s")
    wid = s * NC + c
    zeros16 = jnp.zeros((16,), jnp.float32)
    ones16 = jnp.ones((16,), jnp.float32)

    def zero_hist(i, carry):
        hist_v[pl.ds(i * 16, 16)] = zeros16
        return carry

    lax.fori_loop(0, N_PAD // 16, zero_hist, 0)

    pltpu.sync_copy(dst_hbm.at[pl.ds(wid * EPT, EPT)], dst_v)

    def count(i, carry):
        idx = dst_v[pl.ds(i * 16, 16)]
        plsc.addupdate_scatter(hist_v, [idx], ones16)
        return carry

    lax.fori_loop(0, EPT // 16, count, 0)

    pltpu.sync_copy(hist_v, spart.at[s])
    plsc.subcore_barrier()

    col = s * ROWS_PER_TILE

    def zero_acc(i, carry):
        acc_v[pl.ds(i * 16, 16)] = zeros16
        return carry

    lax.fori_loop(0, ROWS_PER_TILE // 16, zero_acc, 0)

    for j in range(NS):
        pltpu.sync_copy(spart.at[j, pl.ds(col, ROWS_PER_TILE)], tmp_v)

        def add(i, carry):
            acc_v[pl.ds(i * 16, 16)] = (
                acc_v[pl.ds(i * 16, 16)] + tmp_v[pl.ds(i * 16, 16)]
            )
            return carry

        lax.fori_loop(0, ROWS_PER_TILE // 16, add, 0)

    pltpu.sync_copy(acc_v, out_hbm.at[c, pl.ds(col, ROWS_PER_TILE)])


def _make_edge_agg(d, chunk, nch0, nch1, nb):
    assert nch0 % nb == 0 and nch1 % nb == 0

    @functools.partial(
        pl.kernel,
        mesh=_MESH,
        out_type=jax.ShapeDtypeStruct((NC, N_PAD, d), jnp.float32),
        scratch_types=[
            pltpu.VMEM((nch0, chunk), jnp.int32),
            pltpu.VMEM((nch0, chunk), jnp.int32),
            pltpu.VMEM((nb * chunk, d), jnp.float32),
            pltpu.VMEM_SHARED((N_PAD, d), jnp.float32),
            [pltpu.SemaphoreType.DMA] * nb,
            [pltpu.SemaphoreType.DMA] * nb,
        ],
        compiler_params=_SC_PARAMS,
    )
    def edge_agg(g_hbm, src_hbm, dst_hbm, out_hbm,
                 sidx, didx, rows_all, acc, gsem, ssem):
        rows = [rows_all.at[pl.ds(b * chunk, chunk)] for b in range(nb)]
        c = lax.axis_index("c")
        s = lax.axis_index("s")
        row0 = s * ROWS_PER_TILE
        nch_here = jnp.where(c == 0, nch0, nch1)
        nsup = jnp.where(c == 0, nch0 // nb, nch1 // nb)

        pltpu.sync_copy(src_hbm.at[c, s], sidx)
        pltpu.sync_copy(dst_hbm.at[c, s], didx)

        zeros16 = jnp.zeros((16,), jnp.float32)

        def zrow(r, carry):
            for k in range(d // 16):
                rows_all[r, pl.ds(k * 16, 16)] = zeros16
            return carry

        lax.fori_loop(0, chunk, zrow, 0)
        for j in range(ROWS_PER_TILE // chunk):
            pltpu.sync_copy(rows[0], acc.at[pl.ds(row0 + j * chunk, chunk)])

        plsc.subcore_barrier()

        for b in range(nb):
            pltpu.async_copy(g_hbm.at[sidx.at[b]], rows[b], gsem[b])

        def sup(i, carry):
            cc0 = i * nb
            for b in range(nb):
                pltpu.make_async_copy(
                    g_hbm.at[sidx.at[cc0 + b]], rows[b], gsem[b]
                ).wait()
                pltpu.make_async_copy(
                    rows[b], acc.at[didx.at[cc0 + b]], ssem[b]
                ).start(add=True)
            for b in range(nb):
                pltpu.make_async_copy(
                    rows[b], acc.at[didx.at[cc0 + b]], ssem[b]
                ).wait()

                @pl.when(cc0 + b + nb < nch_here)
                def _():
                    pltpu.async_copy(
                        g_hbm.at[sidx.at[cc0 + b + nb]], rows[b], gsem[b]
                    )

            return carry

        lax.fori_loop(0, nsup, sup, 0)

        plsc.subcore_barrier()
        pltpu.sync_copy(
            acc.at[pl.ds(row0, ROWS_PER_TILE)],
            out_hbm.at[c, pl.ds(row0, ROWS_PER_TILE)],
        )

    return edge_agg


_HID_SPLIT = (64, 208, 112)
_LAT_SPLIT = (128, 100, 60)
_edge_agg_hid = _make_edge_agg(D_HID, *_HID_SPLIT, nb=2)
_edge_agg_lat = _make_edge_agg(D_LAT, *_LAT_SPLIT, nb=4)


def _split_edges(flat, chunk, nch0, nch1):
    e0 = NS * nch0 * chunk
    p0 = flat[:e0].reshape(NS, nch0, chunk)
    p1 = flat[e0:].reshape(NS, nch1, chunk)
    p1 = jnp.pad(p1, ((0, 0), (0, nch0 - nch1), (0, 0)), constant_values=N)
    return jnp.stack([p0, p1])


_BLK = 1024
_GRID = N_PAD // _BLK


def _scale_mm_body(hist_ref, x_ref, w_ref, g_ref, dinv_ref):
    deg = hist_ref[:, 0:1] + hist_ref[:, 1:2] + 1.0
    dinv = lax.rsqrt(deg)
    dinv_ref[...] = dinv
    g_ref[...] = jnp.dot(
        x_ref[...] * dinv, w_ref[...], preferred_element_type=jnp.float32
    )


def _scale_mm(hist_t, xp, W1):
    return pl.pallas_call(
        _scale_mm_body,
        grid=(_GRID,),
        in_specs=[
            pl.BlockSpec((_BLK, NC), lambda i: (i, 0)),
            pl.BlockSpec((_BLK, D_IN), lambda i: (i, 0)),
            pl.BlockSpec((D_IN, D_HID), lambda i: (0, 0)),
        ],
        out_specs=[
            pl.BlockSpec((_BLK, D_HID), lambda i: (i, 0)),
            pl.BlockSpec((_BLK, 1), lambda i: (i, 0)),
        ],
        out_shape=[
            jax.ShapeDtypeStruct((N_PAD, D_HID), jnp.float32),
            jax.ShapeDtypeStruct((N_PAD, 1), jnp.float32),
        ],
    )(hist_t, xp, W1)


def _mid_body(sa_ref, sb_ref, g_ref, dinv_ref, b_ref, w_ref, out_ref):
    dinv = dinv_ref[...]
    z = (sa_ref[...] + sb_ref[...] + g_ref[...]) * dinv + b_ref[...]
    z = jnp.maximum(z, 0.0)
    out_ref[...] = (
        jnp.dot(z, w_ref[...], preferred_element_type=jnp.float32) * dinv
    )


def _mid_layer(sa, sb, g1, dinv, b1, W2):
    return pl.pallas_call(
        _mid_body,
        grid=(_GRID,),
        in_specs=[
            pl.BlockSpec((_BLK, D_HID), lambda i: (i, 0)),
            pl.BlockSpec((_BLK, D_HID), lambda i: (i, 0)),
            pl.BlockSpec((_BLK, D_HID), lambda i: (i, 0)),
            pl.BlockSpec((_BLK, 1), lambda i: (i, 0)),
            pl.BlockSpec((1, D_HID), lambda i: (0, 0)),
            pl.BlockSpec((D_HID, D_LAT), lambda i: (0, 0)),
        ],
        out_specs=pl.BlockSpec((_BLK, D_LAT), lambda i: (i, 0)),
        out_shape=jax.ShapeDtypeStruct((N_PAD, D_LAT), jnp.float32),
    )(sa, sb, g1, dinv, b1, W2)


def _final_body(sa_ref, sb_ref, g_ref, dinv_ref, b_ref, out_ref):
    out_ref[...] = (
        (sa_ref[...] + sb_ref[...] + g_ref[...]) * dinv_ref[...] + b_ref[...]
    )


def _final_layer(sa, sb, g2, dinv, b2):
    return pl.pallas_call(
        _final_body,
        grid=(_GRID,),
        in_specs=[
            pl.BlockSpec((_BLK, D_LAT), lambda i: (i, 0)),
            pl.BlockSpec((_BLK, D_LAT), lambda i: (i, 0)),
            pl.BlockSpec((_BLK, D_LAT), lambda i: (i, 0)),
            pl.BlockSpec((_BLK, 1), lambda i: (i, 0)),
            pl.BlockSpec((1, D_LAT), lambda i: (0, 0)),
        ],
        out_specs=pl.BlockSpec((_BLK, D_LAT), lambda i: (i, 0)),
        out_shape=jax.ShapeDtypeStruct((N_PAD, D_LAT), jnp.float32),
    )(sa, sb, g2, dinv, b2)


def kernel(x, edge_index, W1, b1, W2, b2):
    src = edge_index[0]
    dst = edge_index[1]
    pad = jnp.full((E_PAD - E,), N, dtype=jnp.int32)
    srcp = jnp.concatenate([src, pad])
    dstp = jnp.concatenate([dst, pad])
    src_h = _split_edges(srcp, *_HID_SPLIT)
    dst_h = _split_edges(dstp, *_HID_SPLIT)
    src_l = _split_edges(srcp, *_LAT_SPLIT)
    dst_l = _split_edges(dstp, *_LAT_SPLIT)
    xp = jnp.pad(x, ((0, N_PAD - N), (0, 0)))

    hist = _deg_kernel(dstp)
    g1, dinv = _scale_mm(hist.T, xp, W1)
    s1 = _edge_agg_hid(g1, src_h, dst_h)
    g2 = _mid_layer(s1[0], s1[1], g1, dinv, b1.reshape(1, D_HID), W2)
    s2 = _edge_agg_lat(g2, src_l, dst_l)
    out = _final_layer(s2[0], s2[1], g2, dinv, b2.reshape(1, D_LAT))
    return out[:N]


# speedup vs baseline: 1.3443x; 1.3443x over previous
import functools

import jax
import jax.numpy as jnp
from jax import lax
from jax.experimental import pallas as pl
from jax.experimental.pallas import tpu as pltpu
from jax.experimental.pallas import tpu_sc as plsc

N = 10000
E = 320000
D_IN = 128
D_HID = 128
D_LAT = 64

NC = 2
NS = 16
NW = NC * NS

N_PAD = 10240
ROWS_PER_TILE = N_PAD // NS
EPT = 10240
E_PAD = EPT * NW

_MESH = plsc.VectorSubcoreMesh(core_axis_name="c", subcore_axis_name="s")
_SC_PARAMS = pltpu.CompilerParams(
    needs_layout_passes=False, use_tc_tiling_on_sc=False
)


@functools.partial(
    pl.kernel,
    mesh=_MESH,
    out_type=jax.ShapeDtypeStruct((NC, N_PAD), jnp.float32),
    scratch_types=[
        pltpu.VMEM((EPT,), jnp.int32),
        pltpu.VMEM((N_PAD,), jnp.float32),
        pltpu.VMEM((ROWS_PER_TILE,), jnp.float32),
        pltpu.VMEM((ROWS_PER_TILE,), jnp.float32),
        pltpu.VMEM_SHARED((NS, N_PAD), jnp.float32),
    ],
    compiler_params=_SC_PARAMS,
)
def _deg_kernel(dst_hbm, out_hbm, dst_v, hist_v, tmp_v, acc_v, spart):
    c = lax.axis_index("c")
    s = lax.axis_index("s")
    wid = s * NC + c
    zeros16 = jnp.zeros((16,), jnp.float32)
    ones16 = jnp.ones((16,), jnp.float32)

    def zero_hist(i, carry):
        hist_v[pl.ds(i * 16, 16)] = zeros16
        return carry

    lax.fori_loop(0, N_PAD // 16, zero_hist, 0)

    pltpu.sync_copy(dst_hbm.at[pl.ds(wid * EPT, EPT)], dst_v)

    def count(i, carry):
        idx = dst_v[pl.ds(i * 16, 16)]
        plsc.addupdate_scatter(hist_v, [idx], ones16)
        return carry

    lax.fori_loop(0, EPT // 16, count, 0)

    pltpu.sync_copy(hist_v, spart.at[s])
    plsc.subcore_barrier()

    col = s * ROWS_PER_TILE

    def zero_acc(i, carry):
        acc_v[pl.ds(i * 16, 16)] = zeros16
        return carry

    lax.fori_loop(0, ROWS_PER_TILE // 16, zero_acc, 0)

    for j in range(NS):
        pltpu.sync_copy(spart.at[j, pl.ds(col, ROWS_PER_TILE)], tmp_v)

        def add(i, carry):
            acc_v[pl.ds(i * 16, 16)] = (
                acc_v[pl.ds(i * 16, 16)] + tmp_v[pl.ds(i * 16, 16)]
            )
            return carry

        lax.fori_loop(0, ROWS_PER_TILE // 16, add, 0)

    pltpu.sync_copy(acc_v, out_hbm.at[c, pl.ds(col, ROWS_PER_TILE)])


def _make_edge_agg(d, chunk, nch0, nch1, nb):
    assert nch0 % nb == 0 and nch1 % nb == 0

    @functools.partial(
        pl.kernel,
        mesh=_MESH,
        out_type=jax.ShapeDtypeStruct((NC, N_PAD, d), jnp.float32),
        scratch_types=[
            pltpu.VMEM((nch0, chunk), jnp.int32),
            pltpu.VMEM((nch0, chunk), jnp.int32),
            pltpu.VMEM((nb * chunk, d), jnp.float32),
            pltpu.VMEM_SHARED((N_PAD, d), jnp.float32),
            [pltpu.SemaphoreType.DMA] * nb,
            [pltpu.SemaphoreType.DMA] * nb,
        ],
        compiler_params=_SC_PARAMS,
    )
    def edge_agg(g_hbm, src_hbm, dst_hbm, out_hbm,
                 sidx, didx, rows_all, acc, gsem, ssem):
        rows = [rows_all.at[pl.ds(b * chunk, chunk)] for b in range(nb)]
        c = lax.axis_index("c")
        s = lax.axis_index("s")
        row0 = s * ROWS_PER_TILE
        nch_here = jnp.where(c == 0, nch0, nch1)
        nsup = jnp.where(c == 0, nch0 // nb, nch1 // nb)

        pltpu.sync_copy(src_hbm.at[c, s], sidx)
        pltpu.sync_copy(dst_hbm.at[c, s], didx)

        zeros16 = jnp.zeros((16,), jnp.float32)

        def zrow(r, carry):
            for k in range(d // 16):
                rows_all[r, pl.ds(k * 16, 16)] = zeros16
            return carry

        lax.fori_loop(0, chunk, zrow, 0)
        for j in range(ROWS_PER_TILE // chunk):
            pltpu.sync_copy(rows[0], acc.at[pl.ds(row0 + j * chunk, chunk)])

        plsc.subcore_barrier()

        for b in range(nb):
            pltpu.async_copy(g_hbm.at[sidx.at[b]], rows[b], gsem[b])

        def sup(i, carry):
            cc0 = i * nb
            for b in range(nb):
                pltpu.make_async_copy(
                    g_hbm.at[sidx.at[cc0 + b]], rows[b], gsem[b]
                ).wait()
                pltpu.make_async_copy(
                    rows[b], acc.at[didx.at[cc0 + b]], ssem[b]
                ).start(add=True)
            for b in range(nb):
                pltpu.make_async_copy(
                    rows[b], acc.at[didx.at[cc0 + b]], ssem[b]
                ).wait()

                @pl.when(cc0 + b + nb < nch_here)
                def _():
                    pltpu.async_copy(
                        g_hbm.at[sidx.at[cc0 + b + nb]], rows[b], gsem[b]
                    )

            return carry

        lax.fori_loop(0, nsup, sup, 0)

        plsc.subcore_barrier()
        pltpu.sync_copy(
            acc.at[pl.ds(row0, ROWS_PER_TILE)],
            out_hbm.at[c, pl.ds(row0, ROWS_PER_TILE)],
        )

    return edge_agg


def _make_edge_agg_staged(d, chunk, nch0, nch1, nb):
    assert nch0 % nb == 0 and nch1 % nb == 0

    @functools.partial(
        pl.kernel,
        mesh=_MESH,
        out_type=jax.ShapeDtypeStruct((NC, N_PAD, d), jnp.bfloat16),
        scratch_types=[
            pltpu.VMEM((nch0, chunk), jnp.int32),
            pltpu.VMEM((nch0, chunk), jnp.int32),
            pltpu.VMEM((nb * chunk, d), jnp.bfloat16),
            pltpu.VMEM_SHARED((N_PAD, d), jnp.bfloat16),
            pltpu.VMEM_SHARED((N_PAD, d), jnp.bfloat16),
            [pltpu.SemaphoreType.DMA] * nb,
            [pltpu.SemaphoreType.DMA] * nb,
        ],
        compiler_params=_SC_PARAMS,
    )
    def edge_agg(g_hbm, z_hbm, src_hbm, dst_hbm, out_hbm,
                 sidx, didx, rows_all, gbuf, acc, gsem, ssem):
        rows = [rows_all.at[pl.ds(b * chunk, chunk)] for b in range(nb)]
        c = lax.axis_index("c")
        s = lax.axis_index("s")
        row0 = s * ROWS_PER_TILE
        nch_here = jnp.where(c == 0, nch0, nch1)
        nsup = jnp.where(c == 0, nch0 // nb, nch1 // nb)

        pltpu.sync_copy(src_hbm.at[c, s], sidx)
        pltpu.sync_copy(dst_hbm.at[c, s], didx)

        pltpu.sync_copy(
            g_hbm.at[pl.ds(row0, ROWS_PER_TILE)],
            gbuf.at[pl.ds(row0, ROWS_PER_TILE)],
        )
        pltpu.sync_copy(z_hbm, acc.at[pl.ds(row0, ROWS_PER_TILE)])

        plsc.subcore_barrier()

        for b in range(nb):
            pltpu.async_copy(gbuf.at[sidx.at[b]], rows[b], gsem[b])

        def sup(i, carry):
            cc0 = i * nb
            for b in range(nb):
                pltpu.make_async_copy(
                    gbuf.at[sidx.at[cc0 + b]], rows[b], gsem[b]
                ).wait()
                pltpu.make_async_copy(
                    rows[b], acc.at[didx.at[cc0 + b]], ssem[b]
                ).start(add=True)
            for b in range(nb):
                pltpu.make_async_copy(
                    rows[b], acc.at[didx.at[cc0 + b]], ssem[b]
                ).wait()

                @pl.when(cc0 + b + nb < nch_here)
                def _():
                    pltpu.async_copy(
                        gbuf.at[sidx.at[cc0 + b + nb]], rows[b], gsem[b]
                    )

            return carry

        lax.fori_loop(0, nsup, sup, 0)

        plsc.subcore_barrier()
        pltpu.sync_copy(
            acc.at[pl.ds(row0, ROWS_PER_TILE)],
            out_hbm.at[c, pl.ds(row0, ROWS_PER_TILE)],
        )

    return edge_agg


_HID_SPLIT = (64, 208, 112)
_LAT_SPLIT = (128, 80, 80)
_edge_agg_hid = _make_edge_agg(D_HID, *_HID_SPLIT, nb=2)
_edge_agg_lat = _make_edge_agg_staged(D_LAT, *_LAT_SPLIT, nb=4)


def _split_edges(flat, chunk, nch0, nch1):
    e0 = NS * nch0 * chunk
    p0 = flat[:e0].reshape(NS, nch0, chunk)
    p1 = flat[e0:].reshape(NS, nch1, chunk)
    p1 = jnp.pad(p1, ((0, 0), (0, nch0 - nch1), (0, 0)), constant_values=N)
    return jnp.stack([p0, p1])


_BLK = 1024
_GRID = N_PAD // _BLK


def _scale_mm_body(hist_ref, x_ref, w_ref, g_ref, dinv_ref):
    deg = hist_ref[:, 0:1] + hist_ref[:, 1:2] + 1.0
    dinv = lax.rsqrt(deg)
    dinv_ref[...] = dinv
    g_ref[...] = jnp.dot(
        x_ref[...] * dinv, w_ref[...], preferred_element_type=jnp.float32
    )


def _scale_mm(hist_t, xp, W1):
    return pl.pallas_call(
        _scale_mm_body,
        grid=(_GRID,),
        in_specs=[
            pl.BlockSpec((_BLK, NC), lambda i: (i, 0)),
            pl.BlockSpec((_BLK, D_IN), lambda i: (i, 0)),
            pl.BlockSpec((D_IN, D_HID), lambda i: (0, 0)),
        ],
        out_specs=[
            pl.BlockSpec((_BLK, D_HID), lambda i: (i, 0)),
            pl.BlockSpec((_BLK, 1), lambda i: (i, 0)),
        ],
        out_shape=[
            jax.ShapeDtypeStruct((N_PAD, D_HID), jnp.float32),
            jax.ShapeDtypeStruct((N_PAD, 1), jnp.float32),
        ],
    )(hist_t, xp, W1)


def _mid_body(sa_ref, sb_ref, g_ref, dinv_ref, b_ref, w_ref, out_ref, obf_ref):
    dinv = dinv_ref[...]
    z = (sa_ref[...] + sb_ref[...] + g_ref[...]) * dinv + b_ref[...]
    z = jnp.maximum(z, 0.0)
    g2 = jnp.dot(z, w_ref[...], preferred_element_type=jnp.float32) * dinv
    out_ref[...] = g2
    obf_ref[...] = g2.astype(jnp.bfloat16)


def _mid_layer(sa, sb, g1, dinv, b1, W2):
    return pl.pallas_call(
        _mid_body,
        grid=(_GRID,),
        in_specs=[
            pl.BlockSpec((_BLK, D_HID), lambda i: (i, 0)),
            pl.BlockSpec((_BLK, D_HID), lambda i: (i, 0)),
            pl.BlockSpec((_BLK, D_HID), lambda i: (i, 0)),
            pl.BlockSpec((_BLK, 1), lambda i: (i, 0)),
            pl.BlockSpec((1, D_HID), lambda i: (0, 0)),
            pl.BlockSpec((D_HID, D_LAT), lambda i: (0, 0)),
        ],
        out_specs=[
            pl.BlockSpec((_BLK, D_LAT), lambda i: (i, 0)),
            pl.BlockSpec((_BLK, D_LAT), lambda i: (i, 0)),
        ],
        out_shape=[
            jax.ShapeDtypeStruct((N_PAD, D_LAT), jnp.float32),
            jax.ShapeDtypeStruct((N_PAD, D_LAT), jnp.bfloat16),
        ],
    )(sa, sb, g1, dinv, b1, W2)


def _final_body(sa_ref, sb_ref, g_ref, dinv_ref, b_ref, out_ref):
    s = sa_ref[...].astype(jnp.float32) + sb_ref[...].astype(jnp.float32)
    out_ref[...] = (s + g_ref[...]) * dinv_ref[...] + b_ref[...]


def _final_layer(sa, sb, g2, dinv, b2):
    return pl.pallas_call(
        _final_body,
        grid=(_GRID,),
        in_specs=[
            pl.BlockSpec((_BLK, D_LAT), lambda i: (i, 0)),
            pl.BlockSpec((_BLK, D_LAT), lambda i: (i, 0)),
            pl.BlockSpec((_BLK, D_LAT), lambda i: (i, 0)),
            pl.BlockSpec((_BLK, 1), lambda i: (i, 0)),
            pl.BlockSpec((1, D_LAT), lambda i: (0, 0)),
        ],
        out_specs=pl.BlockSpec((_BLK, D_LAT), lambda i: (i, 0)),
        out_shape=jax.ShapeDtypeStruct((N_PAD, D_LAT), jnp.float32),
    )(sa, sb, g2, dinv, b2)


def kernel(x, edge_index, W1, b1, W2, b2):
    src = edge_index[0]
    dst = edge_index[1]
    pad = jnp.full((E_PAD - E,), N, dtype=jnp.int32)
    srcp = jnp.concatenate([src, pad])
    dstp = jnp.concatenate([dst, pad])
    src_h = _split_edges(srcp, *_HID_SPLIT)
    dst_h = _split_edges(dstp, *_HID_SPLIT)
    src_l = _split_edges(srcp, *_LAT_SPLIT)
    dst_l = _split_edges(dstp, *_LAT_SPLIT)
    xp = jnp.pad(x, ((0, N_PAD - N), (0, 0)))

    zlat = jnp.zeros((ROWS_PER_TILE, D_LAT), jnp.bfloat16)

    hist = _deg_kernel(dstp)
    g1, dinv = _scale_mm(hist.T, xp, W1)
    s1 = _edge_agg_hid(g1, src_h, dst_h)
    g2, g2bf = _mid_layer(s1[0], s1[1], g1, dinv, b1.reshape(1, D_HID), W2)
    s2 = _edge_agg_lat(g2bf, zlat, src_l, dst_l)
    out = _final_layer(s2[0], s2[1], g2, dinv, b2.reshape(1, D_LAT))
    return out[:N]


# speedup vs baseline: 2.5528x; 1.8989x over previous
import functools

import jax
import jax.numpy as jnp
from jax import lax
from jax.experimental import pallas as pl
from jax.experimental.pallas import tpu as pltpu
from jax.experimental.pallas import tpu_sc as plsc

N = 10000
E = 320000
D_IN = 128
D_HID = 128
D_LAT = 64

NC = 2
NS = 16
NW = NC * NS

N_PAD = 10240
ROWS_PER_TILE = N_PAD // NS
EPT = 10240
E_PAD = EPT * NW

_MESH = plsc.VectorSubcoreMesh(core_axis_name="c", subcore_axis_name="s")
_SC_PARAMS = pltpu.CompilerParams(
    needs_layout_passes=False, use_tc_tiling_on_sc=False
)


@functools.partial(
    pl.kernel,
    mesh=_MESH,
    out_type=jax.ShapeDtypeStruct((NC, N_PAD), jnp.float32),
    scratch_types=[
        pltpu.VMEM((EPT,), jnp.int32),
        pltpu.VMEM((N_PAD,), jnp.float32),
        pltpu.VMEM((ROWS_PER_TILE,), jnp.float32),
        pltpu.VMEM((ROWS_PER_TILE,), jnp.float32),
        pltpu.VMEM_SHARED((NS, N_PAD), jnp.float32),
    ],
    compiler_params=_SC_PARAMS,
)
def _deg_kernel(dst_hbm, out_hbm, dst_v, hist_v, tmp_v, acc_v, spart):
    c = lax.axis_index("c")
    s = lax.axis_index("s")
    wid = s * NC + c
    zeros16 = jnp.zeros((16,), jnp.float32)
    ones16 = jnp.ones((16,), jnp.float32)

    def zero_hist(i, carry):
        hist_v[pl.ds(i * 16, 16)] = zeros16
        return carry

    lax.fori_loop(0, N_PAD // 16, zero_hist, 0)

    pltpu.sync_copy(dst_hbm.at[pl.ds(wid * EPT, EPT)], dst_v)

    def count(i, carry):
        idx = dst_v[pl.ds(i * 16, 16)]
        plsc.addupdate_scatter(hist_v, [idx], ones16)
        return carry

    lax.fori_loop(0, EPT // 16, count, 0)

    pltpu.sync_copy(hist_v, spart.at[s])
    plsc.subcore_barrier()

    col = s * ROWS_PER_TILE

    def zero_acc(i, carry):
        acc_v[pl.ds(i * 16, 16)] = zeros16
        return carry

    lax.fori_loop(0, ROWS_PER_TILE // 16, zero_acc, 0)

    for j in range(NS):
        pltpu.sync_copy(spart.at[j, pl.ds(col, ROWS_PER_TILE)], tmp_v)

        def add(i, carry):
            acc_v[pl.ds(i * 16, 16)] = (
                acc_v[pl.ds(i * 16, 16)] + tmp_v[pl.ds(i * 16, 16)]
            )
            return carry

        lax.fori_loop(0, ROWS_PER_TILE // 16, add, 0)

    pltpu.sync_copy(acc_v, out_hbm.at[c, pl.ds(col, ROWS_PER_TILE)])


def _make_edge_agg(d, chunk, nch0, nch1, nb):
    assert nch0 % nb == 0 and nch1 % nb == 0

    @functools.partial(
        pl.kernel,
        mesh=_MESH,
        out_type=jax.ShapeDtypeStruct((NC, N_PAD, d), jnp.float32),
        scratch_types=[
            pltpu.VMEM((nch0, chunk), jnp.int32),
            pltpu.VMEM((nch0, chunk), jnp.int32),
            pltpu.VMEM((nb * chunk, d), jnp.float32),
            pltpu.VMEM_SHARED((N_PAD, d), jnp.float32),
            [pltpu.SemaphoreType.DMA] * nb,
            [pltpu.SemaphoreType.DMA] * nb,
        ],
        compiler_params=_SC_PARAMS,
    )
    def edge_agg(g_hbm, src_hbm, dst_hbm, out_hbm,
                 sidx, didx, rows_all, acc, gsem, ssem):
        rows = [rows_all.at[pl.ds(b * chunk, chunk)] for b in range(nb)]
        c = lax.axis_index("c")
        s = lax.axis_index("s")
        row0 = s * ROWS_PER_TILE
        nch_here = jnp.where(c == 0, nch0, nch1)
        nsup = jnp.where(c == 0, nch0 // nb, nch1 // nb)

        pltpu.sync_copy(src_hbm.at[c, s], sidx)
        pltpu.sync_copy(dst_hbm.at[c, s], didx)

        zeros16 = jnp.zeros((16,), jnp.float32)

        def zrow(r, carry):
            for k in range(d // 16):
                rows_all[r, pl.ds(k * 16, 16)] = zeros16
            return carry

        lax.fori_loop(0, chunk, zrow, 0)
        for j in range(ROWS_PER_TILE // chunk):
            pltpu.sync_copy(rows[0], acc.at[pl.ds(row0 + j * chunk, chunk)])

        plsc.subcore_barrier()

        for b in range(nb):
            pltpu.async_copy(g_hbm.at[sidx.at[b]], rows[b], gsem[b])

        def sup(i, carry):
            cc0 = i * nb
            for b in range(nb):
                pltpu.make_async_copy(
                    g_hbm.at[sidx.at[cc0 + b]], rows[b], gsem[b]
                ).wait()
                pltpu.make_async_copy(
                    rows[b], acc.at[didx.at[cc0 + b]], ssem[b]
                ).start(add=True)
            for b in range(nb):
                pltpu.make_async_copy(
                    rows[b], acc.at[didx.at[cc0 + b]], ssem[b]
                ).wait()

                @pl.when(cc0 + b + nb < nch_here)
                def _():
                    pltpu.async_copy(
                        g_hbm.at[sidx.at[cc0 + b + nb]], rows[b], gsem[b]
                    )

            return carry

        lax.fori_loop(0, nsup, sup, 0)

        plsc.subcore_barrier()
        pltpu.sync_copy(
            acc.at[pl.ds(row0, ROWS_PER_TILE)],
            out_hbm.at[c, pl.ds(row0, ROWS_PER_TILE)],
        )

    return edge_agg


def _make_edge_agg_staged(d, chunk, nch0, nch1, nb):
    assert nch0 % nb == 0 and nch1 % nb == 0

    @functools.partial(
        pl.kernel,
        mesh=_MESH,
        out_type=jax.ShapeDtypeStruct((NC, N_PAD, d), jnp.bfloat16),
        scratch_types=[
            pltpu.VMEM((nch0, chunk), jnp.int32),
            pltpu.VMEM((nch0, chunk), jnp.int32),
            pltpu.VMEM((nb * chunk, d), jnp.bfloat16),
            pltpu.VMEM_SHARED((N_PAD, d), jnp.bfloat16),
            pltpu.VMEM_SHARED((N_PAD, d), jnp.bfloat16),
            [pltpu.SemaphoreType.DMA] * nb,
            [pltpu.SemaphoreType.DMA] * nb,
        ],
        compiler_params=_SC_PARAMS,
    )
    def edge_agg(g_hbm, z_hbm, src_hbm, dst_hbm, out_hbm,
                 sidx, didx, rows_all, gbuf, acc, gsem, ssem):
        rows = [rows_all.at[pl.ds(b * chunk, chunk)] for b in range(nb)]
        c = lax.axis_index("c")
        s = lax.axis_index("s")
        row0 = s * ROWS_PER_TILE
        nch_here = jnp.where(c == 0, nch0, nch1)
        nsup = jnp.where(c == 0, nch0 // nb, nch1 // nb)

        pltpu.sync_copy(src_hbm.at[c, s], sidx)
        pltpu.sync_copy(dst_hbm.at[c, s], didx)

        pltpu.sync_copy(
            g_hbm.at[pl.ds(row0, ROWS_PER_TILE)],
            gbuf.at[pl.ds(row0, ROWS_PER_TILE)],
        )
        pltpu.sync_copy(z_hbm, acc.at[pl.ds(row0, ROWS_PER_TILE)])

        plsc.subcore_barrier()

        for b in range(nb):
            pltpu.async_copy(gbuf.at[sidx.at[b]], rows[b], gsem[b])

        def sup(i, carry):
            cc0 = i * nb
            for b in range(nb):
                pltpu.make_async_copy(
                    gbuf.at[sidx.at[cc0 + b]], rows[b], gsem[b]
                ).wait()
                pltpu.make_async_copy(
                    rows[b], acc.at[didx.at[cc0 + b]], ssem[b]
                ).start(add=True)
            for b in range(nb):
                pltpu.make_async_copy(
                    rows[b], acc.at[didx.at[cc0 + b]], ssem[b]
                ).wait()

                @pl.when(cc0 + b + nb < nch_here)
                def _():
                    pltpu.async_copy(
                        gbuf.at[sidx.at[cc0 + b + nb]], rows[b], gsem[b]
                    )

            return carry

        lax.fori_loop(0, nsup, sup, 0)

        plsc.subcore_barrier()
        pltpu.sync_copy(
            acc.at[pl.ds(row0, ROWS_PER_TILE)],
            out_hbm.at[c, pl.ds(row0, ROWS_PER_TILE)],
        )

    return edge_agg


_HID_SPLIT = (64, 160, 160)
_LAT_SPLIT = (128, 80, 80)
_edge_agg_hid = _make_edge_agg_staged(D_HID, *_HID_SPLIT, nb=2)
_edge_agg_lat = _make_edge_agg_staged(D_LAT, *_LAT_SPLIT, nb=4)


def _split_edges(flat, chunk, nch0, nch1):
    e0 = NS * nch0 * chunk
    p0 = flat[:e0].reshape(NS, nch0, chunk)
    p1 = flat[e0:].reshape(NS, nch1, chunk)
    p1 = jnp.pad(p1, ((0, 0), (0, nch0 - nch1), (0, 0)), constant_values=N)
    return jnp.stack([p0, p1])


_BLK = 1024
_GRID = N_PAD // _BLK


def _scale_mm_body(hist_ref, x_ref, w_ref, g_ref, gbf_ref, dinv_ref):
    deg = hist_ref[:, 0:1] + hist_ref[:, 1:2] + 1.0
    dinv = lax.rsqrt(deg)
    dinv_ref[...] = dinv
    g = jnp.dot(
        x_ref[...] * dinv, w_ref[...], preferred_element_type=jnp.float32
    )
    g_ref[...] = g
    gbf_ref[...] = g.astype(jnp.bfloat16)


def _scale_mm(hist_t, xp, W1):
    return pl.pallas_call(
        _scale_mm_body,
        grid=(_GRID,),
        in_specs=[
            pl.BlockSpec((_BLK, NC), lambda i: (i, 0)),
            pl.BlockSpec((_BLK, D_IN), lambda i: (i, 0)),
            pl.BlockSpec((D_IN, D_HID), lambda i: (0, 0)),
        ],
        out_specs=[
            pl.BlockSpec((_BLK, D_HID), lambda i: (i, 0)),
            pl.BlockSpec((_BLK, D_HID), lambda i: (i, 0)),
            pl.BlockSpec((_BLK, 1), lambda i: (i, 0)),
        ],
        out_shape=[
            jax.ShapeDtypeStruct((N_PAD, D_HID), jnp.float32),
            jax.ShapeDtypeStruct((N_PAD, D_HID), jnp.bfloat16),
            jax.ShapeDtypeStruct((N_PAD, 1), jnp.float32),
        ],
    )(hist_t, xp, W1)


def _mid_body(sa_ref, sb_ref, g_ref, dinv_ref, b_ref, w_ref, out_ref, obf_ref):
    dinv = dinv_ref[...]
    s = sa_ref[...].astype(jnp.float32) + sb_ref[...].astype(jnp.float32)
    z = (s + g_ref[...]) * dinv + b_ref[...]
    z = jnp.maximum(z, 0.0)
    g2 = jnp.dot(z, w_ref[...], preferred_element_type=jnp.float32) * dinv
    out_ref[...] = g2
    obf_ref[...] = g2.astype(jnp.bfloat16)


def _mid_layer(sa, sb, g1, dinv, b1, W2):
    return pl.pallas_call(
        _mid_body,
        grid=(_GRID,),
        in_specs=[
            pl.BlockSpec((_BLK, D_HID), lambda i: (i, 0)),
            pl.BlockSpec((_BLK, D_HID), lambda i: (i, 0)),
            pl.BlockSpec((_BLK, D_HID), lambda i: (i, 0)),
            pl.BlockSpec((_BLK, 1), lambda i: (i, 0)),
            pl.BlockSpec((1, D_HID), lambda i: (0, 0)),
            pl.BlockSpec((D_HID, D_LAT), lambda i: (0, 0)),
        ],
        out_specs=[
            pl.BlockSpec((_BLK, D_LAT), lambda i: (i, 0)),
            pl.BlockSpec((_BLK, D_LAT), lambda i: (i, 0)),
        ],
        out_shape=[
            jax.ShapeDtypeStruct((N_PAD, D_LAT), jnp.float32),
            jax.ShapeDtypeStruct((N_PAD, D_LAT), jnp.bfloat16),
        ],
    )(sa, sb, g1, dinv, b1, W2)


def _final_body(sa_ref, sb_ref, g_ref, dinv_ref, b_ref, out_ref):
    s = sa_ref[...].astype(jnp.float32) + sb_ref[...].astype(jnp.float32)
    out_ref[...] = (s + g_ref[...]) * dinv_ref[...] + b_ref[...]


def _final_layer(sa, sb, g2, dinv, b2):
    return pl.pallas_call(
        _final_body,
        grid=(_GRID,),
        in_specs=[
            pl.BlockSpec((_BLK, D_LAT), lambda i: (i, 0)),
            pl.BlockSpec((_BLK, D_LAT), lambda i: (i, 0)),
            pl.BlockSpec((_BLK, D_LAT), lambda i: (i, 0)),
            pl.BlockSpec((_BLK, 1), lambda i: (i, 0)),
            pl.BlockSpec((1, D_LAT), lambda i: (0, 0)),
        ],
        out_specs=pl.BlockSpec((_BLK, D_LAT), lambda i: (i, 0)),
        out_shape=jax.ShapeDtypeStruct((N_PAD, D_LAT), jnp.float32),
    )(sa, sb, g2, dinv, b2)


def kernel(x, edge_index, W1, b1, W2, b2):
    src = edge_index[0]
    dst = edge_index[1]
    pad = jnp.full((E_PAD - E,), N, dtype=jnp.int32)
    srcp = jnp.concatenate([src, pad])
    dstp = jnp.concatenate([dst, pad])
    src_h = _split_edges(srcp, *_HID_SPLIT)
    dst_h = _split_edges(dstp, *_HID_SPLIT)
    src_l = _split_edges(srcp, *_LAT_SPLIT)
    dst_l = _split_edges(dstp, *_LAT_SPLIT)
    xp = jnp.pad(x, ((0, N_PAD - N), (0, 0)))

    zhid = jnp.zeros((ROWS_PER_TILE, D_HID), jnp.bfloat16)
    zlat = jnp.zeros((ROWS_PER_TILE, D_LAT), jnp.bfloat16)

    hist = _deg_kernel(dstp)
    g1, g1bf, dinv = _scale_mm(hist.T, xp, W1)
    s1 = _edge_agg_hid(g1bf, zhid, src_h, dst_h)
    g2, g2bf = _mid_layer(s1[0], s1[1], g1, dinv, b1.reshape(1, D_HID), W2)
    s2 = _edge_agg_lat(g2bf, zlat, src_l, dst_l)
    out = _final_layer(s2[0], s2[1], g2, dinv, b2.reshape(1, D_LAT))
    return out[:N]
